# Initial kernel scaffold; baseline (speedup 1.0000x reference)
import jax
import jax.numpy as jnp
from jax.experimental import pallas as pl


def kernel(target, inp, noise_samples, noise, emb_weight, emb_bias):
    raise NotImplementedError("write your pallas kernel here")


# speedup vs baseline: 2.4699x; 2.4699x over previous
import functools

import jax
import jax.numpy as jnp
import numpy as _np
from jax import lax
from jax.experimental import pallas as pl
from jax.experimental.pallas import tpu as pltpu
from jax.experimental.pallas import tpu_sc as plsc

V = 1000000
D = 64
B = 32
N = 32
K = 100
P = B * N
W = 112
NORM_TERM = 13.815510557964274
MIN_PROB = 1e-9

NC = 2
NS = 16
NW = NC * NS
PB = P // NW
KB = W // 16


def _sc_kernel_body(idx_hbm, inp_hbm, emb_hbm, bias_hbm, noise_hbm,
                    s_out, q_out,
                    idx_v, h_v, rows_v, b_v, q_v, s_v,
                    sem_r, sem_q, sem_b):
    wid = lax.axis_index("s") * NC + lax.axis_index("c")
    base = wid * PB
    pltpu.sync_copy(idx_hbm.at[pl.ds(base, PB)], idx_v)
    pltpu.sync_copy(inp_hbm.at[pl.ds(base, PB)], h_v)

    lane = lax.iota(jnp.int32, 16)

    def pos_body(p, carry):
        cr = pltpu.async_copy(emb_hbm.at[idx_v.at[p]], rows_v, sem_r)
        cq = pltpu.async_copy(noise_hbm.at[idx_v.at[p]], q_v.at[p], sem_q)
        cb = pltpu.async_copy(bias_hbm.at[idx_v.at[p]], b_v, sem_b)
        cr.wait()
        cq.wait()
        cb.wait()

        hs = [h_v[p, pl.ds(j * 16, 16)] for j in range(D // 16)]
        for kb in range(KB):
            svec = jnp.zeros((16,), jnp.float32)
            for j in range(16):
                k = kb * 16 + j
                acc = rows_v[k, pl.ds(0, 16)] * hs[0]
                for t in range(1, D // 16):
                    acc = acc + rows_v[k, pl.ds(t * 16, 16)] * hs[t]
                svec = jnp.where(lane == j, jnp.sum(acc), svec)
            s_v[p, pl.ds(kb * 16, 16)] = svec + b_v[pl.ds(kb * 16, 16)]
        return carry

    lax.fori_loop(0, PB, pos_body, 0)
    pltpu.sync_copy(s_v, s_out.at[pl.ds(base, PB)])
    pltpu.sync_copy(q_v, q_out.at[pl.ds(base, PB)])


def _tc_epilogue_body(s_ref, q_ref, o_ref):
    s = s_ref[...]
    q = q_ref[...]
    p = jnp.clip(jnp.exp(s - NORM_TERM), MIN_PROB, 1.0)
    pt = p / (p + float(K) * q)
    col = lax.broadcasted_iota(jnp.int32, s.shape, 1)
    logp = jnp.maximum(jnp.log(pt), -100.0)
    log1mp = jnp.maximum(jnp.log(1.0 - pt), -100.0)
    bce = jnp.where(col == 0, -logp, -log1mp)
    bce = jnp.where(col < K + 1, bce, 0.0)
    o_ref[...] = (jnp.sum(bce) * (1.0 / P)).reshape(1, 1)


def kernel(target, inp, noise_samples, noise, emb_weight, emb_bias):
    idx = jnp.concatenate(
        [target.reshape(P, 1), noise_samples.reshape(P, K)], axis=1)
    idx = jnp.concatenate(
        [idx, jnp.zeros((P, W - (K + 1)), jnp.int32)], axis=1).astype(jnp.int32)
    inp2d = inp.reshape(P, D).astype(jnp.float32)

    mesh = plsc.VectorSubcoreMesh(core_axis_name="c", subcore_axis_name="s")
    sc = pl.kernel(
        _sc_kernel_body,
        mesh=mesh,
        compiler_params=pltpu.CompilerParams(
            needs_layout_passes=False, use_tc_tiling_on_sc=False),
        out_type=[
            jax.ShapeDtypeStruct((P, W), jnp.float32),
            jax.ShapeDtypeStruct((P, W), jnp.float32),
        ],
        scratch_types=[
            pltpu.VMEM((PB, W), jnp.int32),
            pltpu.VMEM((PB, D), jnp.float32),
            pltpu.VMEM((W, D), jnp.float32),
            pltpu.VMEM((W,), jnp.float32),
            pltpu.VMEM((PB, W), jnp.float32),
            pltpu.VMEM((PB, W), jnp.float32),
            pltpu.SemaphoreType.DMA,
            pltpu.SemaphoreType.DMA,
            pltpu.SemaphoreType.DMA,
        ],
    )
    scores, qvals = sc(idx, inp2d, emb_weight, emb_bias, noise)

    out = pl.pallas_call(
        _tc_epilogue_body,
        out_shape=jax.ShapeDtypeStruct((1, 1), jnp.float32),
    )(scores, qvals)
    return out[0, 0]


# speedup vs baseline: 2.4885x; 1.0075x over previous
import functools

import jax
import jax.numpy as jnp
from jax import lax
from jax.experimental import pallas as pl
from jax.experimental.pallas import tpu as pltpu
from jax.experimental.pallas import tpu_sc as plsc

V = 1000000
D = 64
B = 32
N = 32
K = 100
P = B * N
W = 112
NORM_TERM = 13.815510557964274
MIN_PROB = 1e-9

NC = 2
NS = 16
NW = NC * NS
PB = P // NW
KB = W // 16
NSLOT = 2


def _sc_kernel_body(idx_hbm, inp_hbm, emb_hbm, bias_hbm, noise_hbm,
                    s_out, q_out,
                    idx_v, h_v, rows_v, b_v, q_v, s_v,
                    sem_r, sem_q):
    wid = lax.axis_index("s") * NC + lax.axis_index("c")
    base = wid * PB
    pltpu.sync_copy(idx_hbm.at[pl.ds(base, PB)], idx_v)
    pltpu.sync_copy(inp_hbm.at[pl.ds(base, PB)], h_v)

    lane = lax.iota(jnp.int32, 16)

    for p in range(PB):
        pltpu.async_copy(noise_hbm.at[idx_v.at[p]], q_v.at[p], sem_q)

    def issue_rows(p, slot):
        for kb in range(KB):
            ivec = idx_v[p, pl.ds(kb * 16, 16)]
            for j in range(16):
                r = ivec[j]
                pltpu.async_copy(emb_hbm.at[r], rows_v.at[slot, kb * 16 + j],
                                 sem_r.at[slot])
        pltpu.async_copy(bias_hbm.at[idx_v.at[p]], b_v.at[slot],
                         sem_r.at[slot])

    def wait_rows(slot):
        for k in range(W):
            pltpu.make_async_copy(emb_hbm.at[0], rows_v.at[slot, k],
                                  sem_r.at[slot]).wait()
        pltpu.make_async_copy(bias_hbm.at[pl.ds(0, W)], b_v.at[slot],
                              sem_r.at[slot]).wait()

    issue_rows(0, 0)

    def pos_body(p, carry):
        slot = lax.rem(p, NSLOT)
        nslot = lax.rem(p + 1, NSLOT)
        wait_rows(slot)

        @pl.when(p < PB - 1)
        def _():
            issue_rows(p + 1, nslot)

        hs = [h_v[p, pl.ds(j * 16, 16)] for j in range(D // 16)]
        for kb in range(KB):
            svec = jnp.zeros((16,), jnp.float32)
            for j in range(16):
                k = kb * 16 + j
                acc = rows_v[slot, k, pl.ds(0, 16)] * hs[0]
                for t in range(1, D // 16):
                    acc = acc + rows_v[slot, k, pl.ds(t * 16, 16)] * hs[t]
                svec = jnp.where(lane == j, jnp.sum(acc), svec)
            s_v[p, pl.ds(kb * 16, 16)] = svec + b_v[slot, pl.ds(kb * 16, 16)]
        return carry

    lax.fori_loop(0, PB, pos_body, 0)

    for p in range(PB):
        pltpu.make_async_copy(noise_hbm.at[pl.ds(0, W)], q_v.at[p],
                              sem_q).wait()
    pltpu.sync_copy(s_v, s_out.at[pl.ds(base, PB)])
    pltpu.sync_copy(q_v, q_out.at[pl.ds(base, PB)])


def _tc_epilogue_body(s_ref, q_ref, o_ref):
    s = s_ref[...]
    q = q_ref[...]
    p = jnp.clip(jnp.exp(s - NORM_TERM), MIN_PROB, 1.0)
    pt = p / (p + float(K) * q)
    col = lax.broadcasted_iota(jnp.int32, s.shape, 1)
    logp = jnp.maximum(jnp.log(pt), -100.0)
    log1mp = jnp.maximum(jnp.log(1.0 - pt), -100.0)
    bce = jnp.where(col == 0, -logp, -log1mp)
    bce = jnp.where(col < K + 1, bce, 0.0)
    o_ref[...] = (jnp.sum(bce) * (1.0 / P)).reshape(1, 1)


def kernel(target, inp, noise_samples, noise, emb_weight, emb_bias):
    idx = jnp.concatenate(
        [target.reshape(P, 1), noise_samples.reshape(P, K)], axis=1)
    idx = jnp.concatenate(
        [idx, jnp.zeros((P, W - (K + 1)), jnp.int32)], axis=1).astype(jnp.int32)
    inp2d = inp.reshape(P, D).astype(jnp.float32)

    mesh = plsc.VectorSubcoreMesh(core_axis_name="c", subcore_axis_name="s")
    sc = pl.kernel(
        _sc_kernel_body,
        mesh=mesh,
        compiler_params=pltpu.CompilerParams(
            needs_layout_passes=False, use_tc_tiling_on_sc=True),
        out_type=[
            jax.ShapeDtypeStruct((P, W), jnp.float32),
            jax.ShapeDtypeStruct((P, W), jnp.float32),
        ],
        scratch_types=[
            pltpu.VMEM((PB, W), jnp.int32),
            pltpu.VMEM((PB, D), jnp.float32),
            pltpu.VMEM((NSLOT, W, D), jnp.float32),
            pltpu.VMEM((NSLOT, W), jnp.float32),
            pltpu.VMEM((PB, W), jnp.float32),
            pltpu.VMEM((PB, W), jnp.float32),
            pltpu.SemaphoreType.DMA((NSLOT,)),
            pltpu.SemaphoreType.DMA,
        ],
    )
    scores, qvals = sc(idx, inp2d, emb_weight, emb_bias, noise)

    out = pl.pallas_call(
        _tc_epilogue_body,
        out_shape=jax.ShapeDtypeStruct((1, 1), jnp.float32),
    )(scores, qvals)
    return out[0, 0]
